# Initial kernel scaffold; baseline (speedup 1.0000x reference)
import jax
import jax.numpy as jnp
from jax.experimental import pallas as pl


def kernel(y, lr):
    raise NotImplementedError("write your pallas kernel here")


# speedup vs baseline: 1.0335x; 1.0335x over previous
import jax
import jax.numpy as jnp
from jax.experimental import pallas as pl
from jax.experimental.pallas import tpu as pltpu

_VMEM_LIMIT = 64 * 1024 * 1024


def _rowgroup_kernel(y_ref, lr_ref, mp_ref, mu_ref, o_ref):
    y = y_ref[...]
    pooled = jnp.dot(y, mp_ref[...], preferred_element_type=jnp.float32)
    corr = lr_ref[...] / pooled
    up = jnp.dot(corr, mu_ref[...], preferred_element_type=jnp.float32)
    o_ref[...] = (y * up).astype(o_ref.dtype)


def _pick_block_rows(rows: int, target: int) -> int:
    best = 1
    d = 1
    while d * d <= rows:
        if rows % d == 0:
            for cand in (d, rows // d):
                if cand <= target and cand > best:
                    best = cand
        d += 1
    return best


def kernel(y, lr):
    k = 4
    N, C, H, W = y.shape
    h, w = H // k, W // k
    R = N * C * h
    L = k * W

    y2 = y.reshape(R, L)
    lr2 = lr.reshape(R, w)

    lane = jnp.arange(L)
    member = ((lane % W) // k)[:, None] == jnp.arange(w)[None, :]
    m_pool = member.astype(jnp.float32) * (1.0 / (k * k))
    m_up = member.T.astype(jnp.float32)

    itemsize = jnp.dtype(y.dtype).itemsize
    bt = _pick_block_rows(R, max(8, (2 * 1024 * 1024) // (L * itemsize)))
    grid = (R // bt,)

    out2 = pl.pallas_call(
        _rowgroup_kernel,
        out_shape=jax.ShapeDtypeStruct((R, L), y.dtype),
        grid_spec=pltpu.PrefetchScalarGridSpec(
            num_scalar_prefetch=0,
            grid=grid,
            in_specs=[
                pl.BlockSpec((bt, L), lambda i: (i, 0)),
                pl.BlockSpec((bt, w), lambda i: (i, 0)),
                pl.BlockSpec((L, w), lambda i: (0, 0)),
                pl.BlockSpec((w, L), lambda i: (0, 0)),
            ],
            out_specs=pl.BlockSpec((bt, L), lambda i: (i, 0)),
        ),
        compiler_params=pltpu.CompilerParams(
            dimension_semantics=("parallel",),
            vmem_limit_bytes=_VMEM_LIMIT,
        ),
    )(y2, lr2, m_pool, m_up)

    return out2.reshape(N, C, H, W)


# speedup vs baseline: 1.8026x; 1.7441x over previous
import jax
import jax.numpy as jnp
from jax.experimental import pallas as pl
from jax.experimental.pallas import tpu as pltpu

_VMEM_LIMIT = 64 * 1024 * 1024
_K = 4


def _pool_kernel(y_ref, lr_ref, mp_ref, mu_ref, o_ref):
    _, bc, H, W = y_ref.shape
    _, _, h, w = lr_ref.shape
    k = H // h
    y = y_ref[...].reshape(bc * h, k, W)
    rowsum = y[:, 0, :]
    for r in range(1, k):
        rowsum = rowsum + y[:, r, :]
    pooled = jnp.dot(rowsum, mp_ref[...],
                     preferred_element_type=jnp.float32)
    corr = lr_ref[...].reshape(bc * h, w) / pooled
    up = jnp.dot(corr, mu_ref[...],
                 preferred_element_type=jnp.float32)
    res = y * up[:, None, :]
    o_ref[...] = res.reshape(o_ref.shape).astype(o_ref.dtype)


def kernel(y, lr):
    k = _K
    N, C, H, W = y.shape
    h, w = H // k, W // k

    col = jnp.arange(W) // k
    member = col[:, None] == jnp.arange(w)[None, :]
    m_pool = member.astype(jnp.float32) * (1.0 / (k * k))
    m_up = member.astype(jnp.float32).T

    bc = C
    grid = (N, C // bc)

    out = pl.pallas_call(
        _pool_kernel,
        out_shape=jax.ShapeDtypeStruct((N, C, H, W), y.dtype),
        grid_spec=pltpu.PrefetchScalarGridSpec(
            num_scalar_prefetch=0,
            grid=grid,
            in_specs=[
                pl.BlockSpec((1, bc, H, W), lambda i, j: (i, j, 0, 0)),
                pl.BlockSpec((1, bc, h, w), lambda i, j: (i, j, 0, 0)),
                pl.BlockSpec((W, w), lambda i, j: (0, 0)),
                pl.BlockSpec((w, W), lambda i, j: (0, 0)),
            ],
            out_specs=pl.BlockSpec((1, bc, H, W), lambda i, j: (i, j, 0, 0)),
        ),
        compiler_params=pltpu.CompilerParams(
            dimension_semantics=("parallel", "parallel"),
            vmem_limit_bytes=_VMEM_LIMIT,
        ),
    )(y, lr, m_pool, m_up)

    return out


# speedup vs baseline: 2.0303x; 1.1263x over previous
import functools

import jax
import jax.numpy as jnp
from jax.experimental import pallas as pl
from jax.experimental.pallas import tpu as pltpu

_VMEM_LIMIT = 64 * 1024 * 1024
_K = 4


def _pool_kernel(y_ref, lr_ref, o_ref, *, k):
    bn, bc, H, W = y_ref.shape
    h, w = H // k, W // k
    rows = bn * bc * h

    col = jax.lax.broadcasted_iota(jnp.int32, (W, w), 0) // k
    cell = jax.lax.broadcasted_iota(jnp.int32, (W, w), 1)
    member = (col == cell).astype(jnp.float32)
    m_pool = member * (1.0 / (k * k))

    y = y_ref[...].reshape(rows, k, W)
    rowsum = y[:, 0, :]
    for r in range(1, k):
        rowsum = rowsum + y[:, r, :]
    pooled = jnp.dot(rowsum, m_pool,
                     preferred_element_type=jnp.float32)
    corr = lr_ref[...].reshape(rows, w) / pooled
    up = jnp.dot(corr, member.T,
                 preferred_element_type=jnp.float32)
    res = y * up[:, None, :]
    o_ref[...] = res.reshape(o_ref.shape).astype(o_ref.dtype)


def kernel(y, lr):
    k = _K
    N, C, H, W = y.shape
    h, w = H // k, W // k

    bn, bc = 2, C
    grid = (N // bn, C // bc)

    out = pl.pallas_call(
        functools.partial(_pool_kernel, k=k),
        out_shape=jax.ShapeDtypeStruct((N, C, H, W), y.dtype),
        grid_spec=pltpu.PrefetchScalarGridSpec(
            num_scalar_prefetch=0,
            grid=grid,
            in_specs=[
                pl.BlockSpec((bn, bc, H, W), lambda i, j: (i, j, 0, 0)),
                pl.BlockSpec((bn, bc, h, w), lambda i, j: (i, j, 0, 0)),
            ],
            out_specs=pl.BlockSpec((bn, bc, H, W), lambda i, j: (i, j, 0, 0)),
        ),
        compiler_params=pltpu.CompilerParams(
            dimension_semantics=("parallel", "parallel"),
            vmem_limit_bytes=_VMEM_LIMIT,
        ),
    )(y, lr)

    return out


# speedup vs baseline: 2.0696x; 1.0194x over previous
import functools

import jax
import jax.numpy as jnp
from jax.experimental import pallas as pl
from jax.experimental.pallas import tpu as pltpu

_VMEM_LIMIT = 64 * 1024 * 1024
_K = 4


def _pool_kernel(y_ref, lr_ref, o_ref, *, k):
    bn, bc, H, W = y_ref.shape
    h, w = H // k, W // k
    rows = bn * bc * h

    col = jax.lax.broadcasted_iota(jnp.int32, (W, w), 0) // k
    cell = jax.lax.broadcasted_iota(jnp.int32, (W, w), 1)
    member = (col == cell).astype(jnp.float32)
    m_pool = member * (1.0 / (k * k))

    y = y_ref[...].reshape(rows, k, W)
    rowsum = y[:, 0, :]
    for r in range(1, k):
        rowsum = rowsum + y[:, r, :]
    pooled = jnp.dot(rowsum, m_pool,
                     preferred_element_type=jnp.float32)
    corr = lr_ref[...].reshape(rows, w) / pooled
    up = jnp.dot(corr, member.T,
                 preferred_element_type=jnp.float32)
    res = y * up[:, None, :]
    o_ref[...] = res.reshape(o_ref.shape).astype(o_ref.dtype)


def kernel(y, lr):
    k = _K
    N, C, H, W = y.shape
    h, w = H // k, W // k

    bn, bc = 4, C
    grid = (N // bn, C // bc)

    out = pl.pallas_call(
        functools.partial(_pool_kernel, k=k),
        out_shape=jax.ShapeDtypeStruct((N, C, H, W), y.dtype),
        grid_spec=pltpu.PrefetchScalarGridSpec(
            num_scalar_prefetch=0,
            grid=grid,
            in_specs=[
                pl.BlockSpec((bn, bc, H, W), lambda i, j: (i, j, 0, 0)),
                pl.BlockSpec((bn, bc, h, w), lambda i, j: (i, j, 0, 0)),
            ],
            out_specs=pl.BlockSpec((bn, bc, H, W), lambda i, j: (i, j, 0, 0)),
        ),
        compiler_params=pltpu.CompilerParams(
            dimension_semantics=("parallel", "parallel"),
            vmem_limit_bytes=_VMEM_LIMIT,
        ),
    )(y, lr)

    return out
